# Initial kernel scaffold; baseline (speedup 1.0000x reference)
import jax
import jax.numpy as jnp
from jax.experimental import pallas as pl


def kernel(emos, tags_vec, offsets, table, W_hid, b_hid, W_out, b_out):
    raise NotImplementedError("write your pallas kernel here")


# speedup vs baseline: 78.3428x; 78.3428x over previous
import functools

import jax
import jax.numpy as jnp
from jax import lax
from jax.experimental import pallas as pl
from jax.experimental.pallas import tpu as pltpu
from jax.experimental.pallas import tpu_sc as plsc

_B = 16384
_V = 1000000
_D = 128
_H = 64
_C = 2
_T = 819200

_NW = 32
_BW = _B // _NW
_K = 128


def _bag_sums_sc(tags, offsets, table):
    mesh = plsc.VectorSubcoreMesh(core_axis_name="c", subcore_axis_name="s")

    @functools.partial(
        pl.kernel,
        mesh=mesh,
        out_type=jax.ShapeDtypeStruct((_B, _D), jnp.float32),
        compiler_params=pltpu.CompilerParams(needs_layout_passes=False),
        scratch_types=[
            pltpu.VMEM((_BW + 16,), jnp.int32),
            pltpu.VMEM((_K,), jnp.int32),
            pltpu.VMEM((_K, _D), jnp.float32),
            pltpu.VMEM((_BW, _D), jnp.float32),
            pltpu.SemaphoreType.DMA,
        ],
    )
    def k(tags_hbm, offs_hbm, table_hbm, out_hbm, offs_v, idx_v, rows_v,
          sums_v, sem):
        w = lax.axis_index("s") * 2 + lax.axis_index("c")
        base = w * _BW
        pltpu.sync_copy(offs_hbm.at[pl.ds(base, _BW)], offs_v.at[pl.ds(0, _BW)])

        @pl.when(w < _NW - 1)
        def _():
            pltpu.sync_copy(offs_hbm.at[pl.ds(base + _BW, 16)],
                            offs_v.at[pl.ds(_BW, 16)])

        zero = jnp.zeros((16,), jnp.float32)

        def zbody(r, carry):
            for l in range(_D // 16):
                sums_v[r, pl.ds(l * 16, 16)] = zero
            return carry

        lax.fori_loop(0, _BW, zbody, 0)

        s0 = offs_v[pl.ds(0, 16)][0]
        e = jnp.where(w == _NW - 1, _T, offs_v[pl.ds(_BW, 16)][0])
        s_al = (s0 // 8) * 8
        nchunks = (e - s_al + _K - 1) // _K
        lane = lax.iota(jnp.int32, 16)

        def chunk(ci, carry):
            p = s_al + ci * _K
            p2 = jnp.minimum(p, _T - _K)
            pltpu.sync_copy(tags_hbm.at[pl.ds(p2, _K)], idx_v)
            pltpu.async_copy(table_hbm.at[idx_v], rows_v, sem).wait()
            lo = jnp.maximum(p, s0)

            def group(g, carry):
                tb = p2 + g * 16
                tvec = tb + lane
                pos = jnp.zeros((16,), jnp.int32)
                step = _BW // 2
                while step >= 1:
                    mid = jnp.minimum(pos + step, _BW - 1)
                    ov = plsc.load_gather(offs_v, [mid])
                    pos = jnp.where(ov <= tvec, mid, pos)
                    step //= 2
                for r in range(16):
                    cur = pos[r]

                    @pl.when((tb + r >= lo) & (tb + r < e))
                    def _():
                        for l in range(_D // 16):
                            sl = pl.ds(l * 16, 16)
                            sums_v[cur, sl] = (sums_v[cur, sl]
                                               + rows_v[g * 16 + r, sl])

                return carry

            return lax.fori_loop(0, _K // 16, group, carry)

        lax.fori_loop(0, nchunks, chunk, 0)
        pltpu.sync_copy(sums_v, out_hbm.at[pl.ds(base, _BW)])

    return k(tags, offsets, table)


def _head_tc(sums, offs_cur, offs_next, emos, w_a, w_b, b_hid, w_out, b_out):
    rb = 1024
    grid = (_B // rb,)

    def body(sums_ref, oc_ref, on_ref, emos_ref, wa_ref, wb_ref, bh_ref,
             wo_ref, bo_ref, out_ref):
        cnt = (on_ref[...] - oc_ref[...]).astype(jnp.float32)
        mean = sums_ref[...] * (1.0 / jnp.maximum(cnt, 1.0))
        tf = jnp.maximum(mean, 0.0)
        ew = emos_ref[...]
        hb = jnp.dot(tf, wa_ref[...], preferred_element_type=jnp.float32)
        hb = hb + ew[:, 0:1] * wb_ref[0:1, :] + ew[:, 1:2] * wb_ref[1:2, :]
        h = jnp.maximum(hb + bh_ref[...], 0.0)
        out_ref[...] = jnp.dot(h, wo_ref[...],
                               preferred_element_type=jnp.float32) + bo_ref[...]

    full = lambda shape: pl.BlockSpec(shape, lambda i: (0, 0))
    return pl.pallas_call(
        body,
        grid=grid,
        in_specs=[
            pl.BlockSpec((rb, _D), lambda i: (i, 0)),
            pl.BlockSpec((rb, 1), lambda i: (i, 0)),
            pl.BlockSpec((rb, 1), lambda i: (i, 0)),
            pl.BlockSpec((rb, _C), lambda i: (i, 0)),
            full((_D, _H)),
            full((2, _H)),
            full((1, _H)),
            full((_H, _C)),
            full((1, _C)),
        ],
        out_specs=pl.BlockSpec((rb, _C), lambda i: (i, 0)),
        out_shape=jax.ShapeDtypeStruct((_B, _C), jnp.float32),
    )(sums, offs_cur, offs_next, emos, w_a, w_b, b_hid, w_out, b_out)


def kernel(emos, tags_vec, offsets, table, W_hid, b_hid, W_out, b_out):
    tags_i = tags_vec.astype(jnp.int32)
    offs_i = offsets.astype(jnp.int32)
    sums = _bag_sums_sc(tags_i, offs_i, table)
    offs_next = jnp.concatenate(
        [offs_i[1:], jnp.full((1,), _T, jnp.int32)]).reshape(_B, 1)
    return _head_tc(
        sums,
        offs_i.reshape(_B, 1),
        offs_next,
        emos,
        W_hid[:_D],
        W_hid[_D:],
        b_hid.reshape(1, _H),
        W_out,
        b_out.reshape(1, _C),
    )


# speedup vs baseline: 94.5811x; 1.2073x over previous
import functools

import jax
import jax.numpy as jnp
from jax import lax
from jax.experimental import pallas as pl
from jax.experimental.pallas import tpu as pltpu
from jax.experimental.pallas import tpu_sc as plsc

_B = 16384
_V = 1000000
_D = 128
_H = 64
_C = 2
_T = 819200

_NW = 32
_BW = _B // _NW
_K = 128


def _bag_sums_sc(tags, offsets, table):
    mesh = plsc.VectorSubcoreMesh(core_axis_name="c", subcore_axis_name="s")

    @functools.partial(
        pl.kernel,
        mesh=mesh,
        out_type=jax.ShapeDtypeStruct((_B, _D), jnp.float32),
        compiler_params=pltpu.CompilerParams(needs_layout_passes=False),
        scratch_types=[
            pltpu.VMEM((_BW + 16,), jnp.int32),
            pltpu.VMEM((_K,), jnp.int32),
            pltpu.VMEM((_K,), jnp.int32),
            pltpu.VMEM((_K, _D), jnp.float32),
            pltpu.VMEM((_K, _D), jnp.float32),
            pltpu.VMEM((_BW, _D), jnp.float32),
            pltpu.SemaphoreType.DMA,
            pltpu.SemaphoreType.DMA,
        ],
    )
    def k(tags_hbm, offs_hbm, table_hbm, out_hbm, offs_v, idx0_v, idx1_v,
          rows0_v, rows1_v, sums_v, sem0, sem1):
        w = lax.axis_index("s") * 2 + lax.axis_index("c")
        base = w * _BW
        pltpu.sync_copy(offs_hbm.at[pl.ds(base, _BW)], offs_v.at[pl.ds(0, _BW)])

        @pl.when(w < _NW - 1)
        def _():
            pltpu.sync_copy(offs_hbm.at[pl.ds(base + _BW, 16)],
                            offs_v.at[pl.ds(_BW, 16)])

        zero = jnp.zeros((16,), jnp.float32)

        def zbody(r, carry):
            for l in range(_D // 16):
                sums_v[r, pl.ds(l * 16, 16)] = zero
            return carry

        lax.fori_loop(0, _BW, zbody, 0)

        s0 = offs_v[pl.ds(0, 16)][0]
        e = jnp.where(w == _NW - 1, _T, offs_v[pl.ds(_BW, 16)][0])
        s_al = (s0 // 8) * 8
        nchunks = (e - s_al + _K - 1) // _K
        lane = lax.iota(jnp.int32, 16)

        bufs = ((idx0_v, rows0_v, sem0), (idx1_v, rows1_v, sem1))

        def start(ci, b):
            idx_ref, rows_ref, sem_ref = bufs[b]

            @pl.when(ci < nchunks)
            def _():
                p = s_al + ci * _K
                p2 = jnp.minimum(p, _T - _K)
                pltpu.sync_copy(tags_hbm.at[pl.ds(p2, _K)], idx_ref)
                pltpu.async_copy(table_hbm.at[idx_ref], rows_ref, sem_ref)

        def process(ci, b):
            idx_ref, rows_ref, sem_ref = bufs[b]

            @pl.when(ci < nchunks)
            def _():
                pltpu.make_async_copy(table_hbm.at[idx_ref], rows_ref,
                                      sem_ref).wait()
                p = s_al + ci * _K
                p2 = jnp.minimum(p, _T - _K)
                lo = jnp.maximum(p, s0)

                def group(g, carry):
                    tb = p2 + g * 16
                    tvec = tb + lane
                    pos = jnp.zeros((16,), jnp.int32)
                    step = _BW // 2
                    while step >= 1:
                        mid = jnp.minimum(pos + step, _BW - 1)
                        ov = plsc.load_gather(offs_v, [mid])
                        pos = jnp.where(ov <= tvec, mid, pos)
                        step //= 2
                    for r in range(16):
                        cur = pos[r]

                        @pl.when((tb + r >= lo) & (tb + r < e))
                        def _():
                            for l in range(_D // 16):
                                sl = pl.ds(l * 16, 16)
                                sums_v[cur, sl] = (sums_v[cur, sl]
                                                   + rows_ref[g * 16 + r, sl])

                    return carry

                lax.fori_loop(0, _K // 16, group, 0)

        start(0, 0)
        npairs = (nchunks + 1) // 2

        def pair(pi, carry):
            ci = 2 * pi
            start(ci + 1, 1)
            process(ci, 0)
            start(ci + 2, 0)
            process(ci + 1, 1)
            return carry

        lax.fori_loop(0, npairs, pair, 0)
        pltpu.sync_copy(sums_v, out_hbm.at[pl.ds(base, _BW)])

    return k(tags, offsets, table)


def _head_tc(sums, offs_cur, offs_next, emos, w_a, w_b, b_hid, w_out, b_out):
    rb = 1024
    grid = (_B // rb,)

    def body(sums_ref, oc_ref, on_ref, emos_ref, wa_ref, wb_ref, bh_ref,
             wo_ref, bo_ref, out_ref):
        cnt = (on_ref[...] - oc_ref[...]).astype(jnp.float32)
        mean = sums_ref[...] * (1.0 / jnp.maximum(cnt, 1.0))
        tf = jnp.maximum(mean, 0.0)
        ew = emos_ref[...]
        hb = jnp.dot(tf, wa_ref[...], preferred_element_type=jnp.float32)
        hb = hb + ew[:, 0:1] * wb_ref[0:1, :] + ew[:, 1:2] * wb_ref[1:2, :]
        h = jnp.maximum(hb + bh_ref[...], 0.0)
        out_ref[...] = jnp.dot(h, wo_ref[...],
                               preferred_element_type=jnp.float32) + bo_ref[...]

    full = lambda shape: pl.BlockSpec(shape, lambda i: (0, 0))
    return pl.pallas_call(
        body,
        grid=grid,
        in_specs=[
            pl.BlockSpec((rb, _D), lambda i: (i, 0)),
            pl.BlockSpec((rb, 1), lambda i: (i, 0)),
            pl.BlockSpec((rb, 1), lambda i: (i, 0)),
            pl.BlockSpec((rb, _C), lambda i: (i, 0)),
            full((_D, _H)),
            full((2, _H)),
            full((1, _H)),
            full((_H, _C)),
            full((1, _C)),
        ],
        out_specs=pl.BlockSpec((rb, _C), lambda i: (i, 0)),
        out_shape=jax.ShapeDtypeStruct((_B, _C), jnp.float32),
    )(sums, offs_cur, offs_next, emos, w_a, w_b, b_hid, w_out, b_out)


def kernel(emos, tags_vec, offsets, table, W_hid, b_hid, W_out, b_out):
    tags_i = tags_vec.astype(jnp.int32)
    offs_i = offsets.astype(jnp.int32)
    sums = _bag_sums_sc(tags_i, offs_i, table)
    offs_next = jnp.concatenate(
        [offs_i[1:], jnp.full((1,), _T, jnp.int32)]).reshape(_B, 1)
    return _head_tc(
        sums,
        offs_i.reshape(_B, 1),
        offs_next,
        emos,
        W_hid[:_D],
        W_hid[_D:],
        b_hid.reshape(1, _H),
        W_out,
        b_out.reshape(1, _C),
    )


# speedup vs baseline: 163.1587x; 1.7251x over previous
import functools

import jax
import jax.numpy as jnp
from jax import lax
from jax.experimental import pallas as pl
from jax.experimental.pallas import tpu as pltpu
from jax.experimental.pallas import tpu_sc as plsc

_B = 16384
_V = 1000000
_D = 128
_H = 64
_C = 2
_T = 819200

_NW = 32
_BW = _B // _NW
_K = 128


def _bag_sums_sc(tags, offsets, table):
    mesh = plsc.VectorSubcoreMesh(core_axis_name="c", subcore_axis_name="s")

    @functools.partial(
        pl.kernel,
        mesh=mesh,
        out_type=jax.ShapeDtypeStruct((_B, _D), jnp.float32),
        compiler_params=pltpu.CompilerParams(needs_layout_passes=False),
        scratch_types=[
            pltpu.VMEM((_BW + 16,), jnp.int32),
            pltpu.VMEM((_K,), jnp.int32),
            pltpu.VMEM((_K,), jnp.int32),
            pltpu.VMEM((_K, _D), jnp.float32),
            pltpu.VMEM((_K, _D), jnp.float32),
            pltpu.VMEM((_BW, _D), jnp.float32),
            pltpu.SemaphoreType.DMA,
            pltpu.SemaphoreType.DMA,
        ],
    )
    def k(tags_hbm, offs_hbm, table_hbm, out_hbm, offs_v, idx0_v, idx1_v,
          rows0_v, rows1_v, sums_v, sem0, sem1):
        w = lax.axis_index("s") * 2 + lax.axis_index("c")
        base = w * _BW
        pltpu.sync_copy(offs_hbm.at[pl.ds(base, _BW)], offs_v.at[pl.ds(0, _BW)])

        @pl.when(w < _NW - 1)
        def _():
            pltpu.sync_copy(offs_hbm.at[pl.ds(base + _BW, 16)],
                            offs_v.at[pl.ds(_BW, 16)])

        zero = jnp.zeros((16,), jnp.float32)

        def zbody(r, carry):
            for l in range(_D // 16):
                sums_v[r, pl.ds(l * 16, 16)] = zero
            return carry

        lax.fori_loop(0, _BW, zbody, 0)

        s0 = offs_v[pl.ds(0, 16)][0]
        e = jnp.where(w == _NW - 1, _T, offs_v[pl.ds(_BW, 16)][0])
        s_al = (s0 // 8) * 8
        nchunks = (e - s_al + _K - 1) // _K
        lane = lax.iota(jnp.int32, 16)

        bufs = ((idx0_v, rows0_v, sem0), (idx1_v, rows1_v, sem1))

        def start(ci, b):
            idx_ref, rows_ref, sem_ref = bufs[b]

            @pl.when(ci < nchunks)
            def _():
                p = s_al + ci * _K
                p2 = jnp.minimum(p, _T - _K)
                pltpu.sync_copy(tags_hbm.at[pl.ds(p2, _K)], idx_ref)
                pltpu.async_copy(table_hbm.at[idx_ref], rows_ref, sem_ref)

        def process(ci, b):
            idx_ref, rows_ref, sem_ref = bufs[b]

            @pl.when(ci < nchunks)
            def _():
                pltpu.make_async_copy(table_hbm.at[idx_ref], rows_ref,
                                      sem_ref).wait()
                p = s_al + ci * _K
                p2 = jnp.minimum(p, _T - _K)
                lo = jnp.maximum(p, s0)

                def group(g, carry):
                    tb = p2 + g * 16
                    tvec = tb + lane
                    pos = jnp.zeros((16,), jnp.int32)
                    step = _BW // 2
                    while step >= 1:
                        mid = jnp.minimum(pos + step, _BW - 1)
                        ov = plsc.load_gather(offs_v, [mid])
                        pos = jnp.where(ov <= tvec, mid, pos)
                        step //= 2
                    p0 = pos[0]
                    p15 = pos[15]
                    single = ((p0 == p15) & (tb >= lo) & (tb + 15 < e))

                    @pl.when(single)
                    def _():
                        for l in range(_D // 16):
                            sl = pl.ds(l * 16, 16)
                            v = [rows_ref[g * 16 + r, sl] for r in range(16)]
                            while len(v) > 1:
                                v = [v[2 * i] + v[2 * i + 1]
                                     for i in range(len(v) // 2)]
                            sums_v[p0, sl] = sums_v[p0, sl] + v[0]

                    @pl.when(jnp.logical_not(single))
                    def _():
                        for r in range(16):
                            cur = pos[r]

                            @pl.when((tb + r >= lo) & (tb + r < e))
                            def _():
                                for l in range(_D // 16):
                                    sl = pl.ds(l * 16, 16)
                                    sums_v[cur, sl] = (
                                        sums_v[cur, sl]
                                        + rows_ref[g * 16 + r, sl])

                    return carry

                lax.fori_loop(0, _K // 16, group, 0)

        start(0, 0)
        npairs = (nchunks + 1) // 2

        def pair(pi, carry):
            ci = 2 * pi
            start(ci + 1, 1)
            process(ci, 0)
            start(ci + 2, 0)
            process(ci + 1, 1)
            return carry

        lax.fori_loop(0, npairs, pair, 0)
        pltpu.sync_copy(sums_v, out_hbm.at[pl.ds(base, _BW)])

    return k(tags, offsets, table)


def _head_tc(sums, offs_cur, offs_next, emos, w_a, w_b, b_hid, w_out, b_out):
    rb = 1024
    grid = (_B // rb,)

    def body(sums_ref, oc_ref, on_ref, emos_ref, wa_ref, wb_ref, bh_ref,
             wo_ref, bo_ref, out_ref):
        cnt = (on_ref[...] - oc_ref[...]).astype(jnp.float32)
        mean = sums_ref[...] * (1.0 / jnp.maximum(cnt, 1.0))
        tf = jnp.maximum(mean, 0.0)
        ew = emos_ref[...]
        hb = jnp.dot(tf, wa_ref[...], preferred_element_type=jnp.float32)
        hb = hb + ew[:, 0:1] * wb_ref[0:1, :] + ew[:, 1:2] * wb_ref[1:2, :]
        h = jnp.maximum(hb + bh_ref[...], 0.0)
        out_ref[...] = jnp.dot(h, wo_ref[...],
                               preferred_element_type=jnp.float32) + bo_ref[...]

    full = lambda shape: pl.BlockSpec(shape, lambda i: (0, 0))
    return pl.pallas_call(
        body,
        grid=grid,
        in_specs=[
            pl.BlockSpec((rb, _D), lambda i: (i, 0)),
            pl.BlockSpec((rb, 1), lambda i: (i, 0)),
            pl.BlockSpec((rb, 1), lambda i: (i, 0)),
            pl.BlockSpec((rb, _C), lambda i: (i, 0)),
            full((_D, _H)),
            full((2, _H)),
            full((1, _H)),
            full((_H, _C)),
            full((1, _C)),
        ],
        out_specs=pl.BlockSpec((rb, _C), lambda i: (i, 0)),
        out_shape=jax.ShapeDtypeStruct((_B, _C), jnp.float32),
    )(sums, offs_cur, offs_next, emos, w_a, w_b, b_hid, w_out, b_out)


def kernel(emos, tags_vec, offsets, table, W_hid, b_hid, W_out, b_out):
    tags_i = tags_vec.astype(jnp.int32)
    offs_i = offsets.astype(jnp.int32)
    sums = _bag_sums_sc(tags_i, offs_i, table)
    offs_next = jnp.concatenate(
        [offs_i[1:], jnp.full((1,), _T, jnp.int32)]).reshape(_B, 1)
    return _head_tc(
        sums,
        offs_i.reshape(_B, 1),
        offs_next,
        emos,
        W_hid[:_D],
        W_hid[_D:],
        b_hid.reshape(1, _H),
        W_out,
        b_out.reshape(1, _C),
    )


# speedup vs baseline: 366.6791x; 2.2474x over previous
import functools

import jax
import jax.numpy as jnp
from jax import lax
from jax.experimental import pallas as pl
from jax.experimental.pallas import tpu as pltpu
from jax.experimental.pallas import tpu_sc as plsc

_B = 16384
_V = 1000000
_D = 128
_H = 64
_C = 2
_T = 819200

_NW = 32
_BW = _B // _NW
_K = 128
_NBUF = 3


def _bag_sums_sc(tags, offsets, table):
    mesh = plsc.VectorSubcoreMesh(core_axis_name="c", subcore_axis_name="s")

    @functools.partial(
        pl.kernel,
        mesh=mesh,
        out_type=jax.ShapeDtypeStruct((_B, _D), jnp.float32),
        compiler_params=pltpu.CompilerParams(needs_layout_passes=False),
        scratch_types=[
            pltpu.VMEM((_BW + 16,), jnp.int32),
            [pltpu.VMEM((_K,), jnp.int32) for _ in range(_NBUF)],
            [pltpu.VMEM((_K, _D), jnp.float32) for _ in range(_NBUF)],
            [pltpu.VMEM((_K,), jnp.int32) for _ in range(_NBUF)],
            pltpu.VMEM_SHARED((16 * (_BW + 1), _D), jnp.float32),
            [pltpu.SemaphoreType.DMA for _ in range(_NBUF)],
            [pltpu.SemaphoreType.DMA for _ in range(_NBUF)],
            pltpu.SemaphoreType.DMA,
        ],
    )
    def k(tags_hbm, offs_hbm, table_hbm, out_hbm, offs_v, idxs, rowss, segs,
          shared, gsems, ssems, sem_z):
        sid = lax.axis_index("s")
        w = sid * 2 + lax.axis_index("c")
        base = w * _BW
        sbase = sid * (_BW + 1)

        pltpu.sync_copy(offs_hbm.at[pl.ds(base, _BW)], offs_v.at[pl.ds(0, _BW)])

        @pl.when(w < _NW - 1)
        def _():
            pltpu.sync_copy(offs_hbm.at[pl.ds(base + _BW, 16)],
                            offs_v.at[pl.ds(_BW, 16)])

        zero = jnp.zeros((16,), jnp.float32)

        def zbody(r, carry):
            for l in range(_D // 16):
                rowss[0][r, pl.ds(l * 16, 16)] = zero
            return carry

        lax.fori_loop(0, _K, zbody, 0)
        for j in range(_BW // _K):
            pltpu.async_copy(rowss[0], shared.at[pl.ds(sbase + j * _K, _K)],
                             sem_z).wait()
        pltpu.async_copy(rowss[0].at[pl.ds(0, 1)],
                         shared.at[pl.ds(sbase + _BW, 1)], sem_z).wait()

        s0 = offs_v[pl.ds(0, 16)][0]
        e = jnp.where(w == _NW - 1, _T, offs_v[pl.ds(_BW, 16)][0])
        s_al = (s0 // 8) * 8
        nchunks = (e - s_al + _K - 1) // _K
        lane = lax.iota(jnp.int32, 16)

        def start_gather(ci, b):
            @pl.when(ci < nchunks)
            def _():
                p = s_al + ci * _K
                p2 = jnp.minimum(p, _T - _K)
                pltpu.sync_copy(tags_hbm.at[pl.ds(p2, _K)], idxs[b])
                pltpu.async_copy(table_hbm.at[idxs[b]], rowss[b], gsems[b])

        def wait_scatter(b):
            pltpu.make_async_copy(rowss[b], shared.at[segs[b]],
                                  ssems[b]).wait()

        def slot(ci, b):
            @pl.when(ci < nchunks)
            def _():
                pltpu.make_async_copy(table_hbm.at[idxs[b]], rowss[b],
                                      gsems[b]).wait()
                p = s_al + ci * _K
                p2 = jnp.minimum(p, _T - _K)
                lo = jnp.maximum(p, s0)

                def group(g, carry):
                    tb = p2 + g * 16
                    tvec = tb + lane
                    pos = jnp.zeros((16,), jnp.int32)
                    step = _BW // 2
                    while step >= 1:
                        mid = jnp.minimum(pos + step, _BW - 1)
                        ov = plsc.load_gather(offs_v, [mid])
                        pos = jnp.where(ov <= tvec, mid, pos)
                        step //= 2
                    valid = (tvec >= lo) & (tvec < e)
                    segs[b][pl.ds(g * 16, 16)] = (
                        jnp.where(valid, pos, _BW) + sbase)
                    return carry

                lax.fori_loop(0, _K // 16, group, 0)
                pltpu.async_copy(rowss[b], shared.at[segs[b]], ssems[b],
                                 add=True)

                @pl.when(ci >= 1)
                def _():
                    wait_scatter((b + _NBUF - 1) % _NBUF)

                start_gather(ci + 2, (b + 2) % _NBUF)

        start_gather(0, 0)
        start_gather(1, 1)
        ntriples = (nchunks + _NBUF - 1) // _NBUF

        def triple(pi, carry):
            for b in range(_NBUF):
                slot(_NBUF * pi + b, b)
            return carry

        lax.fori_loop(0, ntriples, triple, 0)
        for b in range(_NBUF):
            @pl.when((nchunks >= 1) & ((nchunks - 1) % _NBUF == b))
            def _():
                wait_scatter(b)

        pltpu.sync_copy(shared.at[pl.ds(sbase, _BW)],
                        out_hbm.at[pl.ds(base, _BW)])

    return k(tags, offsets, table)


def _head_tc(sums, offs_cur, offs_next, emos, w_a, w_b, b_hid, w_out, b_out):
    rb = 1024
    grid = (_B // rb,)

    def body(sums_ref, oc_ref, on_ref, emos_ref, wa_ref, wb_ref, bh_ref,
             wo_ref, bo_ref, out_ref):
        cnt = (on_ref[...] - oc_ref[...]).astype(jnp.float32)
        mean = sums_ref[...] * (1.0 / jnp.maximum(cnt, 1.0))
        tf = jnp.maximum(mean, 0.0)
        ew = emos_ref[...]
        hb = jnp.dot(tf, wa_ref[...], preferred_element_type=jnp.float32)
        hb = hb + ew[:, 0:1] * wb_ref[0:1, :] + ew[:, 1:2] * wb_ref[1:2, :]
        h = jnp.maximum(hb + bh_ref[...], 0.0)
        out_ref[...] = jnp.dot(h, wo_ref[...],
                               preferred_element_type=jnp.float32) + bo_ref[...]

    full = lambda shape: pl.BlockSpec(shape, lambda i: (0, 0))
    return pl.pallas_call(
        body,
        grid=grid,
        in_specs=[
            pl.BlockSpec((rb, _D), lambda i: (i, 0)),
            pl.BlockSpec((rb, 1), lambda i: (i, 0)),
            pl.BlockSpec((rb, 1), lambda i: (i, 0)),
            pl.BlockSpec((rb, _C), lambda i: (i, 0)),
            full((_D, _H)),
            full((2, _H)),
            full((1, _H)),
            full((_H, _C)),
            full((1, _C)),
        ],
        out_specs=pl.BlockSpec((rb, _C), lambda i: (i, 0)),
        out_shape=jax.ShapeDtypeStruct((_B, _C), jnp.float32),
    )(sums, offs_cur, offs_next, emos, w_a, w_b, b_hid, w_out, b_out)


def kernel(emos, tags_vec, offsets, table, W_hid, b_hid, W_out, b_out):
    tags_i = tags_vec.astype(jnp.int32)
    offs_i = offsets.astype(jnp.int32)
    sums = _bag_sums_sc(tags_i, offs_i, table)
    offs_next = jnp.concatenate(
        [offs_i[1:], jnp.full((1,), _T, jnp.int32)]).reshape(_B, 1)
    return _head_tc(
        sums,
        offs_i.reshape(_B, 1),
        offs_next,
        emos,
        W_hid[:_D],
        W_hid[_D:],
        b_hid.reshape(1, _H),
        W_out,
        b_out.reshape(1, _C),
    )


# speedup vs baseline: 417.9377x; 1.1398x over previous
import functools

import jax
import jax.numpy as jnp
from jax import lax
from jax.experimental import pallas as pl
from jax.experimental.pallas import tpu as pltpu
from jax.experimental.pallas import tpu_sc as plsc

_B = 16384
_V = 1000000
_D = 128
_H = 64
_C = 2
_T = 819200

_NW = 32
_BW = _B // _NW
_K = 128
_NBUF = 3


def _bag_sums_sc(tags, offsets, table):
    mesh = plsc.VectorSubcoreMesh(core_axis_name="c", subcore_axis_name="s")

    @functools.partial(
        pl.kernel,
        mesh=mesh,
        out_type=jax.ShapeDtypeStruct((_B, _D), jnp.float32),
        compiler_params=pltpu.CompilerParams(needs_layout_passes=False),
        scratch_types=[
            pltpu.VMEM((_BW + 16,), jnp.int32),
            [pltpu.VMEM((_K,), jnp.int32) for _ in range(_NBUF)],
            [pltpu.VMEM((_K, _D), jnp.float32) for _ in range(_NBUF)],
            [pltpu.VMEM((_K,), jnp.int32) for _ in range(_NBUF)],
            pltpu.VMEM_SHARED((16 * (_BW + 1), _D), jnp.float32),
            [pltpu.SemaphoreType.DMA for _ in range(_NBUF)],
            [pltpu.SemaphoreType.DMA for _ in range(_NBUF)],
            [pltpu.SemaphoreType.DMA for _ in range(_NBUF)],
            pltpu.SemaphoreType.DMA,
        ],
    )
    def k(tags_hbm, offs_hbm, table_hbm, out_hbm, offs_v, idxs, rowss, segs,
          shared, gsems, ssems, isems, sem_z):
        sid = lax.axis_index("s")
        w = sid * 2 + lax.axis_index("c")
        base = w * _BW
        sbase = sid * (_BW + 1)

        pltpu.sync_copy(offs_hbm.at[pl.ds(base, _BW)], offs_v.at[pl.ds(0, _BW)])

        @pl.when(w < _NW - 1)
        def _():
            pltpu.sync_copy(offs_hbm.at[pl.ds(base + _BW, 16)],
                            offs_v.at[pl.ds(_BW, 16)])

        zero = jnp.zeros((16,), jnp.float32)

        def zbody(r, carry):
            for l in range(_D // 16):
                rowss[0][r, pl.ds(l * 16, 16)] = zero
            return carry

        lax.fori_loop(0, _K, zbody, 0)
        for j in range(_BW // _K):
            pltpu.async_copy(rowss[0], shared.at[pl.ds(sbase + j * _K, _K)],
                             sem_z).wait()
        pltpu.async_copy(rowss[0].at[pl.ds(0, 1)],
                         shared.at[pl.ds(sbase + _BW, 1)], sem_z).wait()

        s0 = offs_v[pl.ds(0, 16)][0]
        e = jnp.where(w == _NW - 1, _T, offs_v[pl.ds(_BW, 16)][0])
        s_al = (s0 // 8) * 8
        nchunks = (e - s_al + _K - 1) // _K
        lane = lax.iota(jnp.int32, 16)

        def tag_slice(ci):
            p = s_al + ci * _K
            return jnp.minimum(p, _T - _K)

        def start_idx(ci, b):
            @pl.when(ci < nchunks)
            def _():
                pltpu.async_copy(tags_hbm.at[pl.ds(tag_slice(ci), _K)],
                                 idxs[b], isems[b])

        def start_gather(ci, b):
            @pl.when(ci < nchunks)
            def _():
                pltpu.make_async_copy(tags_hbm.at[pl.ds(tag_slice(ci), _K)],
                                      idxs[b], isems[b]).wait()
                pltpu.async_copy(table_hbm.at[idxs[b]], rowss[b], gsems[b])

        def wait_scatter(b):
            pltpu.make_async_copy(rowss[b], shared.at[segs[b]],
                                  ssems[b]).wait()

        def slot(ci, b):
            @pl.when(ci < nchunks)
            def _():
                pltpu.make_async_copy(table_hbm.at[idxs[b]], rowss[b],
                                      gsems[b]).wait()
                start_idx(ci + _NBUF, b)
                p = s_al + ci * _K
                p2 = jnp.minimum(p, _T - _K)
                lo = jnp.maximum(p, s0)

                def group(g, carry):
                    tb = p2 + g * 16
                    tvec = tb + lane
                    pos = jnp.zeros((16,), jnp.int32)
                    step = _BW // 2
                    while step >= 1:
                        mid = jnp.minimum(pos + step, _BW - 1)
                        ov = plsc.load_gather(offs_v, [mid])
                        pos = jnp.where(ov <= tvec, mid, pos)
                        step //= 2
                    valid = (tvec >= lo) & (tvec < e)
                    segs[b][pl.ds(g * 16, 16)] = (
                        jnp.where(valid, pos, _BW) + sbase)
                    return carry

                lax.fori_loop(0, _K // 16, group, 0)
                pltpu.async_copy(rowss[b], shared.at[segs[b]], ssems[b],
                                 add=True)

                @pl.when(ci >= 1)
                def _():
                    wait_scatter((b + _NBUF - 1) % _NBUF)

                start_gather(ci + 2, (b + 2) % _NBUF)

        for b in range(_NBUF):
            start_idx(b, b)
        start_gather(0, 0)
        start_gather(1, 1)
        ntriples = (nchunks + _NBUF - 1) // _NBUF

        def triple(pi, carry):
            for b in range(_NBUF):
                slot(_NBUF * pi + b, b)
            return carry

        lax.fori_loop(0, ntriples, triple, 0)
        for b in range(_NBUF):
            @pl.when((nchunks >= 1) & ((nchunks - 1) % _NBUF == b))
            def _():
                wait_scatter(b)

        pltpu.sync_copy(shared.at[pl.ds(sbase, _BW)],
                        out_hbm.at[pl.ds(base, _BW)])

    return k(tags, offsets, table)


def _head_tc(sums, offs_cur, offs_next, emos, w_a, w_b, b_hid, w_out, b_out):
    rb = 1024
    grid = (_B // rb,)

    def body(sums_ref, oc_ref, on_ref, emos_ref, wa_ref, wb_ref, bh_ref,
             wo_ref, bo_ref, out_ref):
        cnt = (on_ref[...] - oc_ref[...]).astype(jnp.float32)
        mean = sums_ref[...] * (1.0 / jnp.maximum(cnt, 1.0))
        tf = jnp.maximum(mean, 0.0)
        ew = emos_ref[...]
        hb = jnp.dot(tf, wa_ref[...], preferred_element_type=jnp.float32)
        hb = hb + ew[:, 0:1] * wb_ref[0:1, :] + ew[:, 1:2] * wb_ref[1:2, :]
        h = jnp.maximum(hb + bh_ref[...], 0.0)
        out_ref[...] = jnp.dot(h, wo_ref[...],
                               preferred_element_type=jnp.float32) + bo_ref[...]

    full = lambda shape: pl.BlockSpec(shape, lambda i: (0, 0))
    return pl.pallas_call(
        body,
        grid=grid,
        in_specs=[
            pl.BlockSpec((rb, _D), lambda i: (i, 0)),
            pl.BlockSpec((rb, 1), lambda i: (i, 0)),
            pl.BlockSpec((rb, 1), lambda i: (i, 0)),
            pl.BlockSpec((rb, _C), lambda i: (i, 0)),
            full((_D, _H)),
            full((2, _H)),
            full((1, _H)),
            full((_H, _C)),
            full((1, _C)),
        ],
        out_specs=pl.BlockSpec((rb, _C), lambda i: (i, 0)),
        out_shape=jax.ShapeDtypeStruct((_B, _C), jnp.float32),
    )(sums, offs_cur, offs_next, emos, w_a, w_b, b_hid, w_out, b_out)


def kernel(emos, tags_vec, offsets, table, W_hid, b_hid, W_out, b_out):
    tags_i = tags_vec.astype(jnp.int32)
    offs_i = offsets.astype(jnp.int32)
    sums = _bag_sums_sc(tags_i, offs_i, table)
    offs_next = jnp.concatenate(
        [offs_i[1:], jnp.full((1,), _T, jnp.int32)]).reshape(_B, 1)
    return _head_tc(
        sums,
        offs_i.reshape(_B, 1),
        offs_next,
        emos,
        W_hid[:_D],
        W_hid[_D:],
        b_hid.reshape(1, _H),
        W_out,
        b_out.reshape(1, _C),
    )


# speedup vs baseline: 429.0124x; 1.0265x over previous
import functools

import jax
import jax.numpy as jnp
from jax import lax
from jax.experimental import pallas as pl
from jax.experimental.pallas import tpu as pltpu
from jax.experimental.pallas import tpu_sc as plsc

_B = 16384
_V = 1000000
_D = 128
_H = 64
_C = 2
_T = 819200

_NW = 32
_BW = _B // _NW
_K = 128
_NBUF = 3


def _bag_sums_sc(tags, offsets, table):
    mesh = plsc.VectorSubcoreMesh(core_axis_name="c", subcore_axis_name="s")

    @functools.partial(
        pl.kernel,
        mesh=mesh,
        out_type=jax.ShapeDtypeStruct((_B, _D), jnp.float32),
        compiler_params=pltpu.CompilerParams(needs_layout_passes=False),
        scratch_types=[
            pltpu.VMEM((_BW + 16,), jnp.int32),
            [pltpu.VMEM((_K,), jnp.int32) for _ in range(_NBUF)],
            [pltpu.VMEM((_K, _D), jnp.float32) for _ in range(_NBUF)],
            [pltpu.VMEM((_K,), jnp.int32) for _ in range(_NBUF)],
            pltpu.VMEM_SHARED((16 * (_BW + 1), _D), jnp.float32),
            [pltpu.SemaphoreType.DMA for _ in range(_NBUF)],
            [pltpu.SemaphoreType.DMA for _ in range(_NBUF)],
            [pltpu.SemaphoreType.DMA for _ in range(_NBUF)],
            pltpu.SemaphoreType.DMA,
        ],
    )
    def k(tags_hbm, offs_hbm, table_hbm, out_hbm, offs_v, idxs, rowss, segs,
          shared, gsems, ssems, isems, sem_z):
        sid = lax.axis_index("s")
        w = sid * 2 + lax.axis_index("c")
        base = w * _BW
        sbase = sid * (_BW + 1)

        pltpu.sync_copy(offs_hbm.at[pl.ds(base, _BW)], offs_v.at[pl.ds(0, _BW)])

        @pl.when(w < _NW - 1)
        def _():
            pltpu.sync_copy(offs_hbm.at[pl.ds(base + _BW, 16)],
                            offs_v.at[pl.ds(_BW, 16)])

        s0 = offs_v[pl.ds(0, 16)][0]
        e = jnp.where(w == _NW - 1, _T, offs_v[pl.ds(_BW, 16)][0])
        s_al = (s0 // 8) * 8
        nchunks = (e - s_al + _K - 1) // _K
        lane = lax.iota(jnp.int32, 16)

        def tag_slice(ci):
            p = s_al + ci * _K
            return jnp.minimum(p, _T - _K)

        def start_idx(ci, b):
            @pl.when(ci < nchunks)
            def _():
                pltpu.async_copy(tags_hbm.at[pl.ds(tag_slice(ci), _K)],
                                 idxs[b], isems[b])

        def start_gather(ci, b):
            @pl.when(ci < nchunks)
            def _():
                pltpu.make_async_copy(tags_hbm.at[pl.ds(tag_slice(ci), _K)],
                                      idxs[b], isems[b]).wait()
                pltpu.async_copy(table_hbm.at[idxs[b]], rowss[b], gsems[b])

        def wait_scatter(b):
            pltpu.make_async_copy(rowss[b], shared.at[segs[b]],
                                  ssems[b]).wait()

        def slot(ci, b):
            @pl.when(ci < nchunks)
            def _():
                pltpu.make_async_copy(table_hbm.at[idxs[b]], rowss[b],
                                      gsems[b]).wait()
                start_idx(ci + _NBUF, b)
                p = s_al + ci * _K
                p2 = jnp.minimum(p, _T - _K)
                lo = jnp.maximum(p, s0)

                def group(g, carry):
                    tb = p2 + g * 16
                    tvec = tb + lane
                    pos = jnp.zeros((16,), jnp.int32)
                    step = _BW // 2
                    while step >= 1:
                        mid = jnp.minimum(pos + step, _BW - 1)
                        ov = plsc.load_gather(offs_v, [mid])
                        pos = jnp.where(ov <= tvec, mid, pos)
                        step //= 2
                    valid = (tvec >= lo) & (tvec < e)
                    segs[b][pl.ds(g * 16, 16)] = (
                        jnp.where(valid, pos, _BW) + sbase)
                    return carry

                lax.fori_loop(0, _K // 16, group, 0)
                pltpu.async_copy(rowss[b], shared.at[segs[b]], ssems[b],
                                 add=True)

                @pl.when(ci >= 1)
                def _():
                    wait_scatter((b + _NBUF - 1) % _NBUF)

                start_gather(ci + 2, (b + 2) % _NBUF)

        for b in range(_NBUF):
            start_idx(b, b)
        start_gather(0, 0)
        start_gather(1, 1)

        zero = jnp.zeros((16,), jnp.float32)

        def zbody(r, carry):
            for l in range(_D // 16):
                rowss[2][r, pl.ds(l * 16, 16)] = zero
            return carry

        lax.fori_loop(0, _K, zbody, 0)
        for j in range(_BW // _K):
            pltpu.async_copy(rowss[2], shared.at[pl.ds(sbase + j * _K, _K)],
                             sem_z).wait()
        pltpu.async_copy(rowss[2].at[pl.ds(0, 1)],
                         shared.at[pl.ds(sbase + _BW, 1)], sem_z).wait()

        ntriples = (nchunks + _NBUF - 1) // _NBUF

        def triple(pi, carry):
            for b in range(_NBUF):
                slot(_NBUF * pi + b, b)
            return carry

        lax.fori_loop(0, ntriples, triple, 0)
        for b in range(_NBUF):
            @pl.when((nchunks >= 1) & ((nchunks - 1) % _NBUF == b))
            def _():
                wait_scatter(b)

        pltpu.sync_copy(shared.at[pl.ds(sbase, _BW)],
                        out_hbm.at[pl.ds(base, _BW)])

    return k(tags, offsets, table)


def _head_tc(sums, offs_cur, offs_next, emos, w_a, w_b, b_hid, w_out, b_out):
    rb = 4096
    grid = (_B // rb,)

    def body(sums_ref, oc_ref, on_ref, emos_ref, wa_ref, wb_ref, bh_ref,
             wo_ref, bo_ref, out_ref):
        cnt = (on_ref[...] - oc_ref[...]).astype(jnp.float32)
        mean = sums_ref[...] * (1.0 / jnp.maximum(cnt, 1.0))
        tf = jnp.maximum(mean, 0.0)
        ew = emos_ref[...]
        hb = jnp.dot(tf, wa_ref[...], preferred_element_type=jnp.float32)
        hb = hb + ew[:, 0:1] * wb_ref[0:1, :] + ew[:, 1:2] * wb_ref[1:2, :]
        h = jnp.maximum(hb + bh_ref[...], 0.0)
        out_ref[...] = jnp.dot(h, wo_ref[...],
                               preferred_element_type=jnp.float32) + bo_ref[...]

    full = lambda shape: pl.BlockSpec(shape, lambda i: (0, 0))
    return pl.pallas_call(
        body,
        grid=grid,
        in_specs=[
            pl.BlockSpec((rb, _D), lambda i: (i, 0)),
            pl.BlockSpec((rb, 1), lambda i: (i, 0)),
            pl.BlockSpec((rb, 1), lambda i: (i, 0)),
            pl.BlockSpec((rb, _C), lambda i: (i, 0)),
            full((_D, _H)),
            full((2, _H)),
            full((1, _H)),
            full((_H, _C)),
            full((1, _C)),
        ],
        out_specs=pl.BlockSpec((rb, _C), lambda i: (i, 0)),
        out_shape=jax.ShapeDtypeStruct((_B, _C), jnp.float32),
    )(sums, offs_cur, offs_next, emos, w_a, w_b, b_hid, w_out, b_out)


def kernel(emos, tags_vec, offsets, table, W_hid, b_hid, W_out, b_out):
    tags_i = tags_vec.astype(jnp.int32)
    offs_i = offsets.astype(jnp.int32)
    sums = _bag_sums_sc(tags_i, offs_i, table)
    offs_next = jnp.concatenate(
        [offs_i[1:], jnp.full((1,), _T, jnp.int32)]).reshape(_B, 1)
    return _head_tc(
        sums,
        offs_i.reshape(_B, 1),
        offs_next,
        emos,
        W_hid[:_D],
        W_hid[_D:],
        b_hid.reshape(1, _H),
        W_out,
        b_out.reshape(1, _C),
    )
